# baseline (speedup 1.0000x reference)
import numpy as np

import jax
import jax.numpy as jnp
from jax.experimental import pallas as pl
from jax.experimental.pallas import tpu as pltpu
from jax.sharding import Mesh, PartitionSpec as P

try:
    from jax import shard_map as _shard_map

    def _smap(f, mesh, in_specs, out_specs):
        return _shard_map(
            f, mesh=mesh, in_specs=in_specs, out_specs=out_specs, check_vma=False
        )
except ImportError:
    from jax.experimental.shard_map import shard_map as _shard_map_exp

    def _smap(f, mesh, in_specs, out_specs):
        return _shard_map_exp(
            f, mesh=mesh, in_specs=in_specs, out_specs=out_specs, check_rep=False
        )


def _pick_bm(m):
    for b in (400, 200, 80, 40, 8):
        if m % b == 0:
            return b
    return m


def _gcn_kernel(x_ref, w_ref, adj_ref, out_ref, support_ref):
    @pl.when(pl.program_id(0) == 0)
    def _():
        support_ref[...] = jax.lax.dot_general(
            x_ref[...], w_ref[...],
            dimension_numbers=(((1,), (1,)), ((), ())),
            preferred_element_type=jnp.float32,
        )

    out_ref[...] = jnp.dot(
        adj_ref[...], support_ref[...], preferred_element_type=jnp.float32
    )


def _gcn_local(x, adj, W):
    m, n = adj.shape
    d_in = x.shape[1]
    d_out = W.shape[0]
    bm = _pick_bm(m)
    return pl.pallas_call(
        _gcn_kernel,
        grid=(m // bm,),
        in_specs=[
            pl.BlockSpec((n, d_in), lambda i: (0, 0)),
            pl.BlockSpec((d_out, d_in), lambda i: (0, 0)),
            pl.BlockSpec((bm, n), lambda i: (i, 0)),
        ],
        out_specs=pl.BlockSpec((bm, d_out), lambda i: (i, 0)),
        out_shape=jax.ShapeDtypeStruct((m, d_out), jnp.float32),
        scratch_shapes=[pltpu.VMEM((n, d_out), jnp.float32)],
        compiler_params=pltpu.CompilerParams(
            dimension_semantics=("arbitrary",),
        ),
    )(x, W, adj)


@jax.jit
def kernel(x, adj, W):
    n = adj.shape[0]
    devs = jax.devices()
    nd = len(devs)
    while n % nd:
        nd -= 1
    if nd > 1:
        mesh = Mesh(np.array(devs[:nd]), ("i",))
        f = _smap(
            _gcn_local,
            mesh,
            (P(None, None), P("i", None), P(None, None)),
            P("i", None),
        )
        return f(x, adj, W)
    return _gcn_local(x, adj, W)


# speedup vs baseline: 5.4959x; 5.4959x over previous
import jax
import jax.numpy as jnp
from jax.experimental import pallas as pl
from jax.experimental.pallas import tpu as pltpu

BM = 200


def _gcn_kernel(x_ref, w_ref, adj_ref, out_ref, support_ref):
    @pl.when(pl.program_id(0) == 0)
    def _():
        support_ref[...] = jax.lax.dot_general(
            x_ref[...], w_ref[...],
            dimension_numbers=(((1,), (1,)), ((), ())),
            preferred_element_type=jnp.float32,
        )

    out_ref[...] = jnp.dot(
        adj_ref[...], support_ref[...], preferred_element_type=jnp.float32
    )


@jax.jit
def kernel(x, adj, W):
    n, d_in = x.shape
    d_out = W.shape[0]
    grid = (n // BM,)
    return pl.pallas_call(
        _gcn_kernel,
        grid=grid,
        in_specs=[
            pl.BlockSpec((n, d_in), lambda i: (0, 0)),
            pl.BlockSpec((d_out, d_in), lambda i: (0, 0)),
            pl.BlockSpec((BM, n), lambda i: (i, 0)),
        ],
        out_specs=pl.BlockSpec((BM, d_out), lambda i: (i, 0)),
        out_shape=jax.ShapeDtypeStruct((n, d_out), jnp.float32),
        scratch_shapes=[pltpu.VMEM((n, d_out), jnp.float32)],
        compiler_params=pltpu.CompilerParams(
            dimension_semantics=("arbitrary",),
        ),
    )(x, W, adj)
